# Initial kernel scaffold; baseline (speedup 1.0000x reference)
import jax
import jax.numpy as jnp
from jax.experimental import pallas as pl


def kernel(x, conv1_w, conv1_b, conv2_w, conv2_b, fc1_w, fc1_b, fc2_w, fc2_b, fc3_w, fc3_b):
    raise NotImplementedError("write your pallas kernel here")


# speedup vs baseline: 1.0955x; 1.0955x over previous
import jax
import jax.numpy as jnp
from jax.experimental import pallas as pl
from jax.experimental.pallas import tpu as pltpu

_LANE = 128
_ROWS = 8
_TB = 64


def _ceil_to(v, m):
    return (v + m - 1) // m * m


def _roll_up(v, s):
    if s == 0:
        return v
    return jnp.concatenate([v[s:], v[:s]], axis=0)


def _lenet_body(x_ref, c1w_ref, c1b_ref, c2w_ref, c2b_ref,
                f1w_ref, f1b_ref, f2w_ref, f2b_ref, f3w_ref, f3b_ref,
                o_ref):
    m = x_ref.shape[1]

    s = [x_ref[a] for a in range(4)]
    s = s + [_roll_up(v, 1) for v in s]

    lhs1 = jnp.concatenate(
        [jnp.concatenate([s[o + kh] for kh in range(5)], axis=1)
         for o in range(4)], axis=0)
    acc1 = jnp.dot(lhs1, c1w_ref[...], preferred_element_type=jnp.float32)
    act1 = jnp.maximum(acc1 + c1b_ref[...], 0.0)
    cand1 = jnp.maximum(act1[:, :_LANE], act1[:, _LANE:])
    y0 = jnp.maximum(cand1[:m], cand1[m:2 * m])
    y1 = jnp.maximum(cand1[2 * m:3 * m], cand1[3 * m:])

    t = [y0.astype(jnp.bfloat16), y1.astype(jnp.bfloat16)]
    t = t + [_roll_up(v, 1) for v in t] + [_roll_up(v, 2) for v in t]
    lhs2 = jnp.concatenate(
        [jnp.concatenate([t[di + kh] for kh in range(5)], axis=1)
         for di in range(2)], axis=0)
    acc2 = jnp.dot(lhs2, c2w_ref[...], preferred_element_type=jnp.float32)
    act2 = jnp.maximum(acc2 + c2b_ref[...], 0.0)
    cand2 = jnp.maximum(act2[:, :_LANE], act2[:, _LANE:])
    z = jnp.maximum(cand2[:m], cand2[m:]).astype(jnp.bfloat16)

    lhs3 = jnp.concatenate([_roll_up(z, r) for r in range(5)], axis=1)
    h = jnp.dot(lhs3, f1w_ref[...], preferred_element_type=jnp.float32)
    h = jnp.maximum(h + f1b_ref[...], 0.0).astype(jnp.bfloat16)
    h = jnp.dot(h, f2w_ref[...], preferred_element_type=jnp.float32)
    h = jnp.maximum(h + f2b_ref[...], 0.0).astype(jnp.bfloat16)
    o_ref[...] = (jnp.dot(h, f3w_ref[...], preferred_element_type=jnp.float32)
                  + f3b_ref[...])


def _banded_conv(w, w_in):
    cout, cin, k, _ = w.shape
    ow = (w_in - k + 1) // 2
    kin, kout = w_in * cin, ow * cout
    kin_p, kout_p = _ceil_to(kin, _LANE), _ceil_to(kout, _LANE)
    halves = []
    for dj in range(2):
        kw = jnp.arange(w_in)[:, None] - 2 * jnp.arange(ow)[None, :] - dj
        ok = ((kw >= 0) & (kw < k))[None, None, None]
        v = w[:, :, :, jnp.clip(kw, 0, k - 1)] * ok
        v = jnp.transpose(v, (2, 3, 1, 4, 0)).reshape(k, kin, kout)
        halves.append(jnp.pad(v, ((0, 0), (0, kin_p - kin), (0, kout_p - kout))))
    b = jnp.concatenate(halves, axis=2)
    return b.reshape(k * kin_p, 2 * kout_p).astype(jnp.bfloat16)


def _conv_bias(b, ow, kout_p):
    row = jnp.pad(jnp.tile(b, ow), (0, kout_p - ow * b.shape[0]))
    return jnp.concatenate([row, row]).reshape(1, -1).astype(jnp.float32)


def _fc1_banded(w1, oh, ow, cout):
    d_out = w1.shape[0]
    v = w1.reshape(d_out, cout, oh, ow)
    v = jnp.transpose(v, (2, 3, 1, 0)).reshape(oh, ow * cout, d_out)
    v = jnp.pad(v, ((0, 0), (0, _LANE - ow * cout), (0, _LANE - d_out)))
    return v.reshape(oh * _LANE, _LANE).astype(jnp.bfloat16)


def _fc_mat(w):
    return jnp.pad(w.T, ((0, _LANE - w.shape[1]), (0, _LANE - w.shape[0]))
                   ).astype(jnp.bfloat16)


def _fc_bias(b):
    return jnp.pad(b, (0, _LANE - b.shape[0])).reshape(1, -1).astype(jnp.float32)


def _phase_split(x, bp):
    B, C, H, W = x.shape
    v = jnp.transpose(x, (0, 2, 3, 1)).reshape(B, H // 4, 4, W * C)
    v = jnp.transpose(v, (2, 0, 1, 3))
    v = jnp.pad(v, ((0, 0), (0, bp - B), (0, 0), (0, _LANE - W * C)))
    return v.reshape(4, bp * (H // 4), _LANE).astype(jnp.bfloat16)


def kernel(x, conv1_w, conv1_b, conv2_w, conv2_b,
           fc1_w, fc1_b, fc2_w, fc2_b, fc3_w, fc3_b):
    B, C, H, W = x.shape
    bp = _ceil_to(B, _TB)
    m = _TB * _ROWS

    ow1 = (W - 5 + 1) // 2
    ow2 = (ow1 - 5 + 1) // 2

    xph = _phase_split(x, bp)
    c1w = _banded_conv(conv1_w, W)
    c1b = _conv_bias(conv1_b, ow1, _LANE)
    c2w = _banded_conv(conv2_w, ow1)
    c2b = _conv_bias(conv2_b, ow2, _LANE)
    f1w = _fc1_banded(fc1_w, ow2, ow2, conv2_w.shape[0])
    f1b = _fc_bias(fc1_b)
    f2w = _fc_mat(fc2_w)
    f2b = _fc_bias(fc2_b)
    f3w = _fc_mat(fc3_w)
    f3b = _fc_bias(fc3_b)

    const3 = lambda i: (0, 0)
    out = pl.pallas_call(
        _lenet_body,
        out_shape=jax.ShapeDtypeStruct((bp * _ROWS, _LANE), jnp.float32),
        grid=(bp // _TB,),
        in_specs=[
            pl.BlockSpec((4, m, _LANE), lambda i: (0, i, 0)),
            pl.BlockSpec((5 * _LANE, 2 * _LANE), const3),
            pl.BlockSpec((1, 2 * _LANE), const3),
            pl.BlockSpec((5 * _LANE, 2 * _LANE), const3),
            pl.BlockSpec((1, 2 * _LANE), const3),
            pl.BlockSpec((5 * _LANE, _LANE), const3),
            pl.BlockSpec((1, _LANE), const3),
            pl.BlockSpec((_LANE, _LANE), const3),
            pl.BlockSpec((1, _LANE), const3),
            pl.BlockSpec((_LANE, _LANE), const3),
            pl.BlockSpec((1, _LANE), const3),
        ],
        out_specs=pl.BlockSpec((m, _LANE), lambda i: (i, 0)),
        compiler_params=pltpu.CompilerParams(
            dimension_semantics=("parallel",),
            vmem_limit_bytes=48 * 1024 * 1024),
    )(xph, c1w, c1b, c2w, c2b, f1w, f1b, f2w, f2b, f3w, f3b)

    return out.reshape(bp, _ROWS, _LANE)[:B, 0, :10]


# speedup vs baseline: 1.6042x; 1.4643x over previous
import jax
import jax.numpy as jnp
from jax.experimental import pallas as pl
from jax.experimental.pallas import tpu as pltpu

_LANE = 128
_ROWS = 8
_TB = 64


def _ceil_to(v, m):
    return (v + m - 1) // m * m


def _roll_up(v, s):
    if s == 0:
        return v
    return jnp.concatenate([v[s:], v[:s]], axis=0)


def _lenet_body(x_ref, c1w_ref, c1b_ref, c2w_ref, c2b_ref,
                f1w_ref, f1b_ref, f2w_ref, f2b_ref, f3w_ref, f3b_ref,
                o_ref, z_ref):
    tb = x_ref.shape[0] // 96
    m = tb * _ROWS

    pad = jnp.zeros((m, 32), jnp.bfloat16)
    s = []
    for a in range(4):
        va = x_ref[a::4, :].astype(jnp.bfloat16).reshape(tb, 24, 32)
        parts = [va[:, 8 * c:8 * (c + 1)].reshape(m, 32) for c in range(3)]
        s.append(jnp.concatenate(parts + [pad], axis=1))
    s = s + [_roll_up(v, 1) for v in s]

    lhs1 = jnp.concatenate(
        [jnp.concatenate([s[o + kh] for kh in range(5)], axis=1)
         for o in range(4)], axis=0)
    acc1 = jnp.dot(lhs1, c1w_ref[...], preferred_element_type=jnp.float32)
    act1 = jnp.maximum(acc1 + c1b_ref[...], 0.0)
    cand1 = jnp.maximum(act1[:, :_LANE], act1[:, _LANE:])
    y0 = jnp.maximum(cand1[:m], cand1[m:2 * m])
    y1 = jnp.maximum(cand1[2 * m:3 * m], cand1[3 * m:])

    t = [y0.astype(jnp.bfloat16), y1.astype(jnp.bfloat16)]
    t = t + [_roll_up(v, 1) for v in t] + [_roll_up(v, 2) for v in t]
    lhs2 = jnp.concatenate(
        [jnp.concatenate([t[di + kh] for kh in range(5)], axis=1)
         for di in range(2)], axis=0)
    acc2 = jnp.dot(lhs2, c2w_ref[...], preferred_element_type=jnp.float32)
    act2 = jnp.maximum(acc2 + c2b_ref[...], 0.0)
    cand2 = jnp.maximum(act2[:, :_LANE], act2[:, _LANE:])
    z_ref[...] = jnp.maximum(cand2[:m], cand2[m:])

    lhs3 = jnp.concatenate(
        [z_ref[r::_ROWS, :].astype(jnp.bfloat16) for r in range(5)], axis=1)
    h = jnp.dot(lhs3, f1w_ref[...], preferred_element_type=jnp.float32)
    h = jnp.maximum(h + f1b_ref[...], 0.0).astype(jnp.bfloat16)
    h = jnp.dot(h, f2w_ref[...], preferred_element_type=jnp.float32)
    h = jnp.maximum(h + f2b_ref[...], 0.0).astype(jnp.bfloat16)
    o_ref[...] = (jnp.dot(h, f3w_ref[...], preferred_element_type=jnp.float32)
                  + f3b_ref[...])


def _banded_conv(w, w_in, cmajor):
    cout, cin, k, _ = w.shape
    ow = (w_in - k + 1) // 2
    kin, kout = w_in * cin, ow * cout
    kin_p, kout_p = _ceil_to(kin, _LANE), _ceil_to(kout, _LANE)
    halves = []
    for dj in range(2):
        kw = jnp.arange(w_in)[:, None] - 2 * jnp.arange(ow)[None, :] - dj
        ok = ((kw >= 0) & (kw < k))[None, None, None]
        v = w[:, :, :, jnp.clip(kw, 0, k - 1)] * ok
        perm = (2, 1, 3, 4, 0) if cmajor else (2, 3, 1, 4, 0)
        v = jnp.transpose(v, perm).reshape(k, kin, kout)
        halves.append(jnp.pad(v, ((0, 0), (0, kin_p - kin), (0, kout_p - kout))))
    b = jnp.concatenate(halves, axis=2)
    return b.reshape(k * kin_p, 2 * kout_p).astype(jnp.bfloat16)


def _conv_bias(b, ow, kout_p):
    row = jnp.pad(jnp.tile(b, ow), (0, kout_p - ow * b.shape[0]))
    return jnp.concatenate([row, row]).reshape(1, -1).astype(jnp.float32)


def _fc1_banded(w1, oh, ow, cout):
    d_out = w1.shape[0]
    v = w1.reshape(d_out, cout, oh, ow)
    v = jnp.transpose(v, (2, 3, 1, 0)).reshape(oh, ow * cout, d_out)
    v = jnp.pad(v, ((0, 0), (0, _LANE - ow * cout), (0, _LANE - d_out)))
    return v.reshape(oh * _LANE, _LANE).astype(jnp.bfloat16)


def _fc_mat(w):
    return jnp.pad(w.T, ((0, _LANE - w.shape[1]), (0, _LANE - w.shape[0]))
                   ).astype(jnp.bfloat16)


def _fc_bias(b):
    return jnp.pad(b, (0, _LANE - b.shape[0])).reshape(1, -1).astype(jnp.float32)


def kernel(x, conv1_w, conv1_b, conv2_w, conv2_b,
           fc1_w, fc1_b, fc2_w, fc2_b, fc3_w, fc3_b):
    B, C, H, W = x.shape
    bp = _ceil_to(B, _TB)
    if bp != B:
        x = jnp.pad(x, ((0, bp - B), (0, 0), (0, 0), (0, 0)))
    xr = x.reshape(bp * C * H, W)

    ow1 = (W - 5 + 1) // 2
    ow2 = (ow1 - 5 + 1) // 2

    c1w = _banded_conv(conv1_w, W, cmajor=True)
    c1b = _conv_bias(conv1_b, ow1, _LANE)
    c2w = _banded_conv(conv2_w, ow1, cmajor=False)
    c2b = _conv_bias(conv2_b, ow2, _LANE)
    f1w = _fc1_banded(fc1_w, ow2, ow2, conv2_w.shape[0])
    f1b = _fc_bias(fc1_b)
    f2w = _fc_mat(fc2_w)
    f2b = _fc_bias(fc2_b)
    f3w = _fc_mat(fc3_w)
    f3b = _fc_bias(fc3_b)

    const = lambda i: (0, 0)
    out = pl.pallas_call(
        _lenet_body,
        out_shape=jax.ShapeDtypeStruct((bp, _LANE), jnp.float32),
        grid=(bp // _TB,),
        in_specs=[
            pl.BlockSpec((_TB * C * H, W), lambda i: (i, 0)),
            pl.BlockSpec((5 * _LANE, 2 * _LANE), const),
            pl.BlockSpec((1, 2 * _LANE), const),
            pl.BlockSpec((5 * _LANE, 2 * _LANE), const),
            pl.BlockSpec((1, 2 * _LANE), const),
            pl.BlockSpec((5 * _LANE, _LANE), const),
            pl.BlockSpec((1, _LANE), const),
            pl.BlockSpec((_LANE, _LANE), const),
            pl.BlockSpec((1, _LANE), const),
            pl.BlockSpec((_LANE, _LANE), const),
            pl.BlockSpec((1, _LANE), const),
        ],
        out_specs=pl.BlockSpec((_TB, _LANE), lambda i: (i, 0)),
        scratch_shapes=[pltpu.VMEM((_TB * _ROWS, _LANE), jnp.float32)],
        compiler_params=pltpu.CompilerParams(
            dimension_semantics=("parallel",),
            vmem_limit_bytes=48 * 1024 * 1024),
    )(xr, c1w, c1b, c2w, c2b, f1w, f1b, f2w, f2b, f3w, f3b)

    return out[:B, :10]


# speedup vs baseline: 1.6054x; 1.0008x over previous
import jax
import jax.numpy as jnp
from jax.experimental import pallas as pl
from jax.experimental.pallas import tpu as pltpu

_LANE = 128
_ROWS = 8
_TB = 64


def _ceil_to(v, m):
    return (v + m - 1) // m * m


def _roll_up(v, s):
    if s == 0:
        return v
    return jnp.concatenate([v[s:], v[:s]], axis=0)


def _lenet_body(x_ref, c1w_ref, c1b_ref, c2w_ref, c2b_ref,
                f1w_ref, f1b_ref, f2w_ref, f2b_ref, f3w_ref, f3b_ref,
                o_ref, z_ref):
    tb = x_ref.shape[0] // 96
    m = tb * _ROWS

    pad = jnp.zeros((m, 32), jnp.bfloat16)
    s = []
    for a in range(4):
        va = x_ref[a::4, :].astype(jnp.bfloat16).reshape(tb, 24, 32)
        parts = [va[:, 8 * c:8 * (c + 1)].reshape(m, 32) for c in range(3)]
        s.append(jnp.concatenate(parts + [pad], axis=1))
    s = s + [_roll_up(v, 1) for v in s]

    lhs1 = jnp.concatenate(
        [jnp.concatenate([s[o + kh] for kh in range(5)], axis=1)
         for o in range(4)], axis=0)
    acc1 = jnp.dot(lhs1, c1w_ref[...], preferred_element_type=jnp.float32)
    act1 = jnp.maximum(acc1 + c1b_ref[...], 0.0)
    cand1 = jnp.maximum(act1[:, :_LANE], act1[:, _LANE:])
    y0 = jnp.maximum(cand1[:m], cand1[m:2 * m])
    y1 = jnp.maximum(cand1[2 * m:3 * m], cand1[3 * m:])

    t = [y0.astype(jnp.bfloat16), y1.astype(jnp.bfloat16)]
    t = t + [_roll_up(v, 1) for v in t] + [_roll_up(v, 2) for v in t]
    lhs2 = jnp.concatenate(
        [jnp.concatenate([t[di + kh] for kh in range(5)], axis=1)
         for di in range(2)], axis=0)
    acc2 = jnp.dot(lhs2, c2w_ref[...], preferred_element_type=jnp.float32)
    act2 = jnp.maximum(acc2 + c2b_ref[...], 0.0)
    cand2 = jnp.maximum(act2[:, :_LANE], act2[:, _LANE:])
    z_ref[...] = jnp.maximum(cand2[:m], cand2[m:])

    lhs3 = jnp.concatenate(
        [z_ref[r::_ROWS, :].astype(jnp.bfloat16) for r in range(5)], axis=1)
    h = jnp.dot(lhs3, f1w_ref[...], preferred_element_type=jnp.float32)
    h = jnp.maximum(h + f1b_ref[...], 0.0).astype(jnp.bfloat16)
    h = jnp.dot(h, f2w_ref[...], preferred_element_type=jnp.float32)
    h = jnp.maximum(h + f2b_ref[...], 0.0).astype(jnp.bfloat16)
    logits = (jnp.dot(h, f3w_ref[...], preferred_element_type=jnp.float32)
              + f3b_ref[...])
    o_ref[...] = logits[:, :o_ref.shape[1]]


def _banded_conv(w, w_in, cmajor):
    cout, cin, k, _ = w.shape
    ow = (w_in - k + 1) // 2
    kin, kout = w_in * cin, ow * cout
    kin_p, kout_p = _ceil_to(kin, _LANE), _ceil_to(kout, _LANE)
    halves = []
    for dj in range(2):
        kw = jnp.arange(w_in)[:, None] - 2 * jnp.arange(ow)[None, :] - dj
        ok = ((kw >= 0) & (kw < k))[None, None, None]
        v = w[:, :, :, jnp.clip(kw, 0, k - 1)] * ok
        perm = (2, 1, 3, 4, 0) if cmajor else (2, 3, 1, 4, 0)
        v = jnp.transpose(v, perm).reshape(k, kin, kout)
        halves.append(jnp.pad(v, ((0, 0), (0, kin_p - kin), (0, kout_p - kout))))
    b = jnp.concatenate(halves, axis=2)
    return b.reshape(k * kin_p, 2 * kout_p).astype(jnp.bfloat16)


def _conv_bias(b, ow, kout_p):
    row = jnp.pad(jnp.tile(b, ow), (0, kout_p - ow * b.shape[0]))
    return jnp.concatenate([row, row]).reshape(1, -1).astype(jnp.float32)


def _fc1_banded(w1, oh, ow, cout):
    d_out = w1.shape[0]
    v = w1.reshape(d_out, cout, oh, ow)
    v = jnp.transpose(v, (2, 3, 1, 0)).reshape(oh, ow * cout, d_out)
    v = jnp.pad(v, ((0, 0), (0, _LANE - ow * cout), (0, _LANE - d_out)))
    return v.reshape(oh * _LANE, _LANE).astype(jnp.bfloat16)


def _fc_mat(w):
    return jnp.pad(w.T, ((0, _LANE - w.shape[1]), (0, _LANE - w.shape[0]))
                   ).astype(jnp.bfloat16)


def _fc_bias(b):
    return jnp.pad(b, (0, _LANE - b.shape[0])).reshape(1, -1).astype(jnp.float32)


def kernel(x, conv1_w, conv1_b, conv2_w, conv2_b,
           fc1_w, fc1_b, fc2_w, fc2_b, fc3_w, fc3_b):
    B, C, H, W = x.shape
    bp = _ceil_to(B, _TB)
    if bp != B:
        x = jnp.pad(x, ((0, bp - B), (0, 0), (0, 0), (0, 0)))
    xr = x.reshape(bp * C * H, W)

    ow1 = (W - 5 + 1) // 2
    ow2 = (ow1 - 5 + 1) // 2

    c1w = _banded_conv(conv1_w, W, cmajor=True)
    c1b = _conv_bias(conv1_b, ow1, _LANE)
    c2w = _banded_conv(conv2_w, ow1, cmajor=False)
    c2b = _conv_bias(conv2_b, ow2, _LANE)
    f1w = _fc1_banded(fc1_w, ow2, ow2, conv2_w.shape[0])
    f1b = _fc_bias(fc1_b)
    f2w = _fc_mat(fc2_w)
    f2b = _fc_bias(fc2_b)
    f3w = _fc_mat(fc3_w)
    f3b = _fc_bias(fc3_b)

    const = lambda i: (0, 0)
    out = pl.pallas_call(
        _lenet_body,
        out_shape=jax.ShapeDtypeStruct((bp, 10), jnp.float32),
        grid=(bp // _TB,),
        in_specs=[
            pl.BlockSpec((_TB * C * H, W), lambda i: (i, 0)),
            pl.BlockSpec((5 * _LANE, 2 * _LANE), const),
            pl.BlockSpec((1, 2 * _LANE), const),
            pl.BlockSpec((5 * _LANE, 2 * _LANE), const),
            pl.BlockSpec((1, 2 * _LANE), const),
            pl.BlockSpec((5 * _LANE, _LANE), const),
            pl.BlockSpec((1, _LANE), const),
            pl.BlockSpec((_LANE, _LANE), const),
            pl.BlockSpec((1, _LANE), const),
            pl.BlockSpec((_LANE, _LANE), const),
            pl.BlockSpec((1, _LANE), const),
        ],
        out_specs=pl.BlockSpec((_TB, 10), lambda i: (i, 0)),
        scratch_shapes=[pltpu.VMEM((_TB * _ROWS, _LANE), jnp.float32)],
        compiler_params=pltpu.CompilerParams(
            dimension_semantics=("parallel",),
            vmem_limit_bytes=48 * 1024 * 1024),
    )(xr, c1w, c1b, c2w, c2b, f1w, f1b, f2w, f2b, f3w, f3b)

    return out[:B]


# speedup vs baseline: 1.6408x; 1.0220x over previous
import jax
import jax.numpy as jnp
from jax.experimental import pallas as pl
from jax.experimental.pallas import tpu as pltpu

_LANE = 128
_ROWS = 8
_TB = 64


def _ceil_to(v, m):
    return (v + m - 1) // m * m


def _roll_up(v, s):
    if s == 0:
        return v
    return jnp.concatenate([v[s:], v[:s]], axis=0)


def _lenet_body(x_ref, c1w_ref, c1b_ref, c2w_ref, c2b_ref,
                f1w_ref, f1b_ref, f2w_ref, f2b_ref, f3w_ref, f3b_ref,
                o_ref, z_ref):
    tb = x_ref.shape[0] // 3
    m = tb * _ROWS

    pad = jnp.zeros((m, 32), jnp.bfloat16)
    s = []
    for a in range(4):
        va = x_ref[:, a::4, :].astype(jnp.bfloat16)
        va = va.reshape(tb, 3, _ROWS, 32)
        parts = [va[:, c].reshape(m, 32) for c in range(3)]
        s.append(jnp.concatenate(parts + [pad], axis=1))
    s = s + [_roll_up(v, 1) for v in s]

    lhs1 = jnp.concatenate(
        [jnp.concatenate([s[o + kh] for kh in range(5)], axis=1)
         for o in range(4)], axis=0)
    acc1 = jnp.dot(lhs1, c1w_ref[...], preferred_element_type=jnp.float32)
    act1 = jnp.maximum(acc1 + c1b_ref[...], 0.0)
    cand1 = jnp.maximum(act1[:, :_LANE], act1[:, _LANE:])
    y0 = jnp.maximum(cand1[:m], cand1[m:2 * m])
    y1 = jnp.maximum(cand1[2 * m:3 * m], cand1[3 * m:])

    t = [y0.astype(jnp.bfloat16), y1.astype(jnp.bfloat16)]
    t = t + [_roll_up(v, 1) for v in t] + [_roll_up(v, 2) for v in t]
    lhs2 = jnp.concatenate(
        [jnp.concatenate([t[di + kh] for kh in range(5)], axis=1)
         for di in range(2)], axis=0)
    acc2 = jnp.dot(lhs2, c2w_ref[...], preferred_element_type=jnp.float32)
    act2 = jnp.maximum(acc2 + c2b_ref[...], 0.0)
    cand2 = jnp.maximum(act2[:, :_LANE], act2[:, _LANE:])
    z_ref[...] = jnp.maximum(cand2[:m], cand2[m:])

    lhs3 = jnp.concatenate(
        [z_ref[r::_ROWS, :].astype(jnp.bfloat16) for r in range(5)], axis=1)
    h = jnp.dot(lhs3, f1w_ref[...], preferred_element_type=jnp.float32)
    h = jnp.maximum(h + f1b_ref[...], 0.0).astype(jnp.bfloat16)
    h = jnp.dot(h, f2w_ref[...], preferred_element_type=jnp.float32)
    h = jnp.maximum(h + f2b_ref[...], 0.0).astype(jnp.bfloat16)
    logits = (jnp.dot(h, f3w_ref[...], preferred_element_type=jnp.float32)
              + f3b_ref[...])
    o_ref[...] = logits[:, :o_ref.shape[1]]


def _banded_conv(w, w_in, cmajor):
    cout, cin, k, _ = w.shape
    ow = (w_in - k + 1) // 2
    kin, kout = w_in * cin, ow * cout
    kin_p, kout_p = _ceil_to(kin, _LANE), _ceil_to(kout, _LANE)
    halves = []
    for dj in range(2):
        kw = jnp.arange(w_in)[:, None] - 2 * jnp.arange(ow)[None, :] - dj
        ok = ((kw >= 0) & (kw < k))[None, None, None]
        v = w[:, :, :, jnp.clip(kw, 0, k - 1)] * ok
        perm = (2, 1, 3, 4, 0) if cmajor else (2, 3, 1, 4, 0)
        v = jnp.transpose(v, perm).reshape(k, kin, kout)
        halves.append(jnp.pad(v, ((0, 0), (0, kin_p - kin), (0, kout_p - kout))))
    b = jnp.concatenate(halves, axis=2)
    return b.reshape(k * kin_p, 2 * kout_p).astype(jnp.bfloat16)


def _conv_bias(b, ow, kout_p):
    row = jnp.pad(jnp.tile(b, ow), (0, kout_p - ow * b.shape[0]))
    return jnp.concatenate([row, row]).reshape(1, -1).astype(jnp.float32)


def _fc1_banded(w1, oh, ow, cout):
    d_out = w1.shape[0]
    v = w1.reshape(d_out, cout, oh, ow)
    v = jnp.transpose(v, (2, 3, 1, 0)).reshape(oh, ow * cout, d_out)
    v = jnp.pad(v, ((0, 0), (0, _LANE - ow * cout), (0, _LANE - d_out)))
    return v.reshape(oh * _LANE, _LANE).astype(jnp.bfloat16)


def _fc_mat(w):
    return jnp.pad(w.T, ((0, _LANE - w.shape[1]), (0, _LANE - w.shape[0]))
                   ).astype(jnp.bfloat16)


def _fc_bias(b):
    return jnp.pad(b, (0, _LANE - b.shape[0])).reshape(1, -1).astype(jnp.float32)


def kernel(x, conv1_w, conv1_b, conv2_w, conv2_b,
           fc1_w, fc1_b, fc2_w, fc2_b, fc3_w, fc3_b):
    B, C, H, W = x.shape
    bp = _ceil_to(B, _TB)
    if bp != B:
        x = jnp.pad(x, ((0, bp - B), (0, 0), (0, 0), (0, 0)))
    xr = x.reshape(bp * C, H, W)

    ow1 = (W - 5 + 1) // 2
    ow2 = (ow1 - 5 + 1) // 2

    c1w = _banded_conv(conv1_w, W, cmajor=True)
    c1b = _conv_bias(conv1_b, ow1, _LANE)
    c2w = _banded_conv(conv2_w, ow1, cmajor=False)
    c2b = _conv_bias(conv2_b, ow2, _LANE)
    f1w = _fc1_banded(fc1_w, ow2, ow2, conv2_w.shape[0])
    f1b = _fc_bias(fc1_b)
    f2w = _fc_mat(fc2_w)
    f2b = _fc_bias(fc2_b)
    f3w = _fc_mat(fc3_w)
    f3b = _fc_bias(fc3_b)

    const = lambda i: (0, 0)
    out = pl.pallas_call(
        _lenet_body,
        out_shape=jax.ShapeDtypeStruct((bp, 10), jnp.float32),
        grid=(bp // _TB,),
        in_specs=[
            pl.BlockSpec((_TB * C, H, W), lambda i: (i, 0, 0)),
            pl.BlockSpec((5 * _LANE, 2 * _LANE), const),
            pl.BlockSpec((1, 2 * _LANE), const),
            pl.BlockSpec((5 * _LANE, 2 * _LANE), const),
            pl.BlockSpec((1, 2 * _LANE), const),
            pl.BlockSpec((5 * _LANE, _LANE), const),
            pl.BlockSpec((1, _LANE), const),
            pl.BlockSpec((_LANE, _LANE), const),
            pl.BlockSpec((1, _LANE), const),
            pl.BlockSpec((_LANE, _LANE), const),
            pl.BlockSpec((1, _LANE), const),
        ],
        out_specs=pl.BlockSpec((_TB, 10), lambda i: (i, 0)),
        scratch_shapes=[pltpu.VMEM((_TB * _ROWS, _LANE), jnp.float32)],
        compiler_params=pltpu.CompilerParams(
            dimension_semantics=("parallel",),
            vmem_limit_bytes=48 * 1024 * 1024),
    )(xr, c1w, c1b, c2w, c2b, f1w, f1b, f2w, f2b, f3w, f3b)

    return out[:B]


# speedup vs baseline: 1.8062x; 1.1008x over previous
import jax
import jax.numpy as jnp
from jax.experimental import pallas as pl
from jax.experimental.pallas import tpu as pltpu

_LANE = 128
_ROWS = 8
_TB = 128


def _ceil_to(v, m):
    return (v + m - 1) // m * m


def _roll_up(v, s):
    if s == 0:
        return v
    return jnp.concatenate([v[s:], v[:s]], axis=0)


def _lenet_body(x_ref, c1w_ref, c1b_ref, c2w_ref, c2b_ref,
                f1w_ref, f1b_ref, f2w_ref, f2b_ref, f3w_ref, f3b_ref,
                o_ref, z_ref):
    tb = x_ref.shape[0] // 3
    m = tb * _ROWS

    pad = jnp.zeros((m, 32), jnp.bfloat16)
    s = []
    for a in range(4):
        va = x_ref[:, a::4, :].astype(jnp.bfloat16)
        va = va.reshape(tb, 3, _ROWS, 32)
        parts = [va[:, c].reshape(m, 32) for c in range(3)]
        s.append(jnp.concatenate(parts + [pad], axis=1))
    s = s + [_roll_up(v, 1) for v in s]

    lhs1 = jnp.concatenate(
        [jnp.concatenate([s[o + kh] for kh in range(5)], axis=1)
         for o in range(4)], axis=0)
    acc1 = jnp.dot(lhs1, c1w_ref[...], preferred_element_type=jnp.float32)
    act1 = jnp.maximum(acc1 + c1b_ref[...], 0.0)
    cand1 = jnp.maximum(act1[:, :_LANE], act1[:, _LANE:])
    y0 = jnp.maximum(cand1[:m], cand1[m:2 * m])
    y1 = jnp.maximum(cand1[2 * m:3 * m], cand1[3 * m:])

    t = [y0.astype(jnp.bfloat16), y1.astype(jnp.bfloat16)]
    t = t + [_roll_up(v, 1) for v in t] + [_roll_up(v, 2) for v in t]
    lhs2 = jnp.concatenate(
        [jnp.concatenate([t[di + kh] for kh in range(5)], axis=1)
         for di in range(2)], axis=0)
    acc2 = jnp.dot(lhs2, c2w_ref[...], preferred_element_type=jnp.float32)
    act2 = jnp.maximum(acc2 + c2b_ref[...], 0.0)
    cand2 = jnp.maximum(act2[:, :_LANE], act2[:, _LANE:])
    z_ref[...] = jnp.maximum(cand2[:m], cand2[m:])

    lhs3 = jnp.concatenate(
        [z_ref[r::_ROWS, :].astype(jnp.bfloat16) for r in range(5)], axis=1)
    h = jnp.dot(lhs3, f1w_ref[...], preferred_element_type=jnp.float32)
    h = jnp.maximum(h + f1b_ref[...], 0.0).astype(jnp.bfloat16)
    h = jnp.dot(h, f2w_ref[...], preferred_element_type=jnp.float32)
    h = jnp.maximum(h + f2b_ref[...], 0.0).astype(jnp.bfloat16)
    logits = (jnp.dot(h, f3w_ref[...], preferred_element_type=jnp.float32)
              + f3b_ref[...])
    o_ref[...] = logits[:, :o_ref.shape[1]]


def _banded_conv(w, w_in, cmajor):
    cout, cin, k, _ = w.shape
    ow = (w_in - k + 1) // 2
    kin, kout = w_in * cin, ow * cout
    kin_p, kout_p = _ceil_to(kin, _LANE), _ceil_to(kout, _LANE)
    halves = []
    for dj in range(2):
        kw = jnp.arange(w_in)[:, None] - 2 * jnp.arange(ow)[None, :] - dj
        ok = ((kw >= 0) & (kw < k))[None, None, None]
        v = w[:, :, :, jnp.clip(kw, 0, k - 1)] * ok
        perm = (2, 1, 3, 4, 0) if cmajor else (2, 3, 1, 4, 0)
        v = jnp.transpose(v, perm).reshape(k, kin, kout)
        halves.append(jnp.pad(v, ((0, 0), (0, kin_p - kin), (0, kout_p - kout))))
    b = jnp.concatenate(halves, axis=2)
    return b.reshape(k * kin_p, 2 * kout_p).astype(jnp.bfloat16)


def _conv_bias(b, ow, kout_p):
    row = jnp.pad(jnp.tile(b, ow), (0, kout_p - ow * b.shape[0]))
    return jnp.concatenate([row, row]).reshape(1, -1).astype(jnp.float32)


def _fc1_banded(w1, oh, ow, cout):
    d_out = w1.shape[0]
    v = w1.reshape(d_out, cout, oh, ow)
    v = jnp.transpose(v, (2, 3, 1, 0)).reshape(oh, ow * cout, d_out)
    v = jnp.pad(v, ((0, 0), (0, _LANE - ow * cout), (0, _LANE - d_out)))
    return v.reshape(oh * _LANE, _LANE).astype(jnp.bfloat16)


def _fc_mat(w):
    return jnp.pad(w.T, ((0, _LANE - w.shape[1]), (0, _LANE - w.shape[0]))
                   ).astype(jnp.bfloat16)


def _fc_bias(b):
    return jnp.pad(b, (0, _LANE - b.shape[0])).reshape(1, -1).astype(jnp.float32)


def kernel(x, conv1_w, conv1_b, conv2_w, conv2_b,
           fc1_w, fc1_b, fc2_w, fc2_b, fc3_w, fc3_b):
    B, C, H, W = x.shape
    bp = _ceil_to(B, _TB)
    if bp != B:
        x = jnp.pad(x, ((0, bp - B), (0, 0), (0, 0), (0, 0)))
    xr = x.reshape(bp * C, H, W)

    ow1 = (W - 5 + 1) // 2
    ow2 = (ow1 - 5 + 1) // 2

    c1w = _banded_conv(conv1_w, W, cmajor=True)
    c1b = _conv_bias(conv1_b, ow1, _LANE)
    c2w = _banded_conv(conv2_w, ow1, cmajor=False)
    c2b = _conv_bias(conv2_b, ow2, _LANE)
    f1w = _fc1_banded(fc1_w, ow2, ow2, conv2_w.shape[0])
    f1b = _fc_bias(fc1_b)
    f2w = _fc_mat(fc2_w)
    f2b = _fc_bias(fc2_b)
    f3w = _fc_mat(fc3_w)
    f3b = _fc_bias(fc3_b)

    const = lambda i: (0, 0)
    out = pl.pallas_call(
        _lenet_body,
        out_shape=jax.ShapeDtypeStruct((bp, 10), jnp.float32),
        grid=(bp // _TB,),
        in_specs=[
            pl.BlockSpec((_TB * C, H, W), lambda i: (i, 0, 0)),
            pl.BlockSpec((5 * _LANE, 2 * _LANE), const),
            pl.BlockSpec((1, 2 * _LANE), const),
            pl.BlockSpec((5 * _LANE, 2 * _LANE), const),
            pl.BlockSpec((1, 2 * _LANE), const),
            pl.BlockSpec((5 * _LANE, _LANE), const),
            pl.BlockSpec((1, _LANE), const),
            pl.BlockSpec((_LANE, _LANE), const),
            pl.BlockSpec((1, _LANE), const),
            pl.BlockSpec((_LANE, _LANE), const),
            pl.BlockSpec((1, _LANE), const),
        ],
        out_specs=pl.BlockSpec((_TB, 10), lambda i: (i, 0)),
        scratch_shapes=[pltpu.VMEM((_TB * _ROWS, _LANE), jnp.float32)],
        compiler_params=pltpu.CompilerParams(
            dimension_semantics=("parallel",),
            vmem_limit_bytes=48 * 1024 * 1024),
    )(xr, c1w, c1b, c2w, c2b, f1w, f1b, f2w, f2b, f3w, f3b)

    return out[:B]


# speedup vs baseline: 1.8779x; 1.0397x over previous
import jax
import jax.numpy as jnp
from jax.experimental import pallas as pl
from jax.experimental.pallas import tpu as pltpu

_LANE = 128
_ROWS = 8
_TB = 256


def _ceil_to(v, m):
    return (v + m - 1) // m * m


def _roll_up(v, s):
    if s == 0:
        return v
    return jnp.concatenate([v[s:], v[:s]], axis=0)


def _lenet_body(x_ref, c1w_ref, c1b_ref, c2w_ref, c2b_ref,
                f1w_ref, f1b_ref, f2w_ref, f2b_ref, f3w_ref, f3b_ref,
                o_ref, z_ref):
    tb = x_ref.shape[0] // 3
    m = tb * _ROWS

    pad = jnp.zeros((m, 32), jnp.bfloat16)
    s = []
    for a in range(4):
        va = x_ref[:, a::4, :].astype(jnp.bfloat16)
        va = va.reshape(tb, 3, _ROWS, 32)
        parts = [va[:, c].reshape(m, 32) for c in range(3)]
        s.append(jnp.concatenate(parts + [pad], axis=1))
    s = s + [_roll_up(v, 1) for v in s]

    lhs1 = jnp.concatenate(
        [jnp.concatenate([s[o + kh] for kh in range(5)], axis=1)
         for o in range(4)], axis=0)
    acc1 = jnp.dot(lhs1, c1w_ref[...], preferred_element_type=jnp.float32)
    act1 = jnp.maximum(acc1 + c1b_ref[...], 0.0)
    cand1 = jnp.maximum(act1[:, :_LANE], act1[:, _LANE:])
    y0 = jnp.maximum(cand1[:m], cand1[m:2 * m])
    y1 = jnp.maximum(cand1[2 * m:3 * m], cand1[3 * m:])

    t = [y0.astype(jnp.bfloat16), y1.astype(jnp.bfloat16)]
    t = t + [_roll_up(v, 1) for v in t] + [_roll_up(v, 2) for v in t]
    lhs2 = jnp.concatenate(
        [jnp.concatenate([t[di + kh] for kh in range(5)], axis=1)
         for di in range(2)], axis=0)
    acc2 = jnp.dot(lhs2, c2w_ref[...], preferred_element_type=jnp.float32)
    act2 = jnp.maximum(acc2 + c2b_ref[...], 0.0)
    cand2 = jnp.maximum(act2[:, :_LANE], act2[:, _LANE:])
    z_ref[...] = jnp.maximum(cand2[:m], cand2[m:])

    lhs3 = jnp.concatenate(
        [z_ref[r::_ROWS, :].astype(jnp.bfloat16) for r in range(5)], axis=1)
    h = jnp.dot(lhs3, f1w_ref[...], preferred_element_type=jnp.float32)
    h = jnp.maximum(h + f1b_ref[...], 0.0).astype(jnp.bfloat16)
    h = jnp.dot(h, f2w_ref[...], preferred_element_type=jnp.float32)
    h = jnp.maximum(h + f2b_ref[...], 0.0).astype(jnp.bfloat16)
    logits = (jnp.dot(h, f3w_ref[...], preferred_element_type=jnp.float32)
              + f3b_ref[...])
    o_ref[...] = logits[:, :o_ref.shape[1]]


def _banded_conv(w, w_in, cmajor):
    cout, cin, k, _ = w.shape
    ow = (w_in - k + 1) // 2
    kin, kout = w_in * cin, ow * cout
    kin_p, kout_p = _ceil_to(kin, _LANE), _ceil_to(kout, _LANE)
    halves = []
    for dj in range(2):
        kw = jnp.arange(w_in)[:, None] - 2 * jnp.arange(ow)[None, :] - dj
        ok = ((kw >= 0) & (kw < k))[None, None, None]
        v = w[:, :, :, jnp.clip(kw, 0, k - 1)] * ok
        perm = (2, 1, 3, 4, 0) if cmajor else (2, 3, 1, 4, 0)
        v = jnp.transpose(v, perm).reshape(k, kin, kout)
        halves.append(jnp.pad(v, ((0, 0), (0, kin_p - kin), (0, kout_p - kout))))
    b = jnp.concatenate(halves, axis=2)
    return b.reshape(k * kin_p, 2 * kout_p).astype(jnp.bfloat16)


def _conv_bias(b, ow, kout_p):
    row = jnp.pad(jnp.tile(b, ow), (0, kout_p - ow * b.shape[0]))
    return jnp.concatenate([row, row]).reshape(1, -1).astype(jnp.float32)


def _fc1_banded(w1, oh, ow, cout):
    d_out = w1.shape[0]
    v = w1.reshape(d_out, cout, oh, ow)
    v = jnp.transpose(v, (2, 3, 1, 0)).reshape(oh, ow * cout, d_out)
    v = jnp.pad(v, ((0, 0), (0, _LANE - ow * cout), (0, _LANE - d_out)))
    return v.reshape(oh * _LANE, _LANE).astype(jnp.bfloat16)


def _fc_mat(w):
    return jnp.pad(w.T, ((0, _LANE - w.shape[1]), (0, _LANE - w.shape[0]))
                   ).astype(jnp.bfloat16)


def _fc_bias(b):
    return jnp.pad(b, (0, _LANE - b.shape[0])).reshape(1, -1).astype(jnp.float32)


def kernel(x, conv1_w, conv1_b, conv2_w, conv2_b,
           fc1_w, fc1_b, fc2_w, fc2_b, fc3_w, fc3_b):
    B, C, H, W = x.shape
    bp = _ceil_to(B, _TB)
    if bp != B:
        x = jnp.pad(x, ((0, bp - B), (0, 0), (0, 0), (0, 0)))
    xr = x.reshape(bp * C, H, W)

    ow1 = (W - 5 + 1) // 2
    ow2 = (ow1 - 5 + 1) // 2

    c1w = _banded_conv(conv1_w, W, cmajor=True)
    c1b = _conv_bias(conv1_b, ow1, _LANE)
    c2w = _banded_conv(conv2_w, ow1, cmajor=False)
    c2b = _conv_bias(conv2_b, ow2, _LANE)
    f1w = _fc1_banded(fc1_w, ow2, ow2, conv2_w.shape[0])
    f1b = _fc_bias(fc1_b)
    f2w = _fc_mat(fc2_w)
    f2b = _fc_bias(fc2_b)
    f3w = _fc_mat(fc3_w)
    f3b = _fc_bias(fc3_b)

    const = lambda i: (0, 0)
    out = pl.pallas_call(
        _lenet_body,
        out_shape=jax.ShapeDtypeStruct((bp, 10), jnp.float32),
        grid=(bp // _TB,),
        in_specs=[
            pl.BlockSpec((_TB * C, H, W), lambda i: (i, 0, 0)),
            pl.BlockSpec((5 * _LANE, 2 * _LANE), const),
            pl.BlockSpec((1, 2 * _LANE), const),
            pl.BlockSpec((5 * _LANE, 2 * _LANE), const),
            pl.BlockSpec((1, 2 * _LANE), const),
            pl.BlockSpec((5 * _LANE, _LANE), const),
            pl.BlockSpec((1, _LANE), const),
            pl.BlockSpec((_LANE, _LANE), const),
            pl.BlockSpec((1, _LANE), const),
            pl.BlockSpec((_LANE, _LANE), const),
            pl.BlockSpec((1, _LANE), const),
        ],
        out_specs=pl.BlockSpec((_TB, 10), lambda i: (i, 0)),
        scratch_shapes=[pltpu.VMEM((_TB * _ROWS, _LANE), jnp.float32)],
        compiler_params=pltpu.CompilerParams(
            dimension_semantics=("parallel",),
            vmem_limit_bytes=48 * 1024 * 1024),
    )(xr, c1w, c1b, c2w, c2b, f1w, f1b, f2w, f2b, f3w, f3b)

    return out[:B]


# speedup vs baseline: 2.2559x; 1.2013x over previous
import jax
import jax.numpy as jnp
from jax.experimental import pallas as pl
from jax.experimental.pallas import tpu as pltpu

_LANE = 128
_ROWS = 8
_TB = 256


def _ceil_to(v, m):
    return (v + m - 1) // m * m


def _roll_up(v, s):
    if s == 0:
        return v
    return jnp.concatenate([v[s:], v[:s]], axis=0)


def _lenet_body(x_ref, c1w_ref, c1b_ref, c2w_ref, c2b_ref,
                f1w_ref, f1b_ref, f2w_ref, f2b_ref, f3w_ref, f3b_ref,
                o_ref, scr_ref):
    tb = x_ref.shape[3]
    m = tb * _ROWS

    @pl.when(pl.program_id(0) == 0)
    def _init():
        scr_ref[...] = jnp.zeros(scr_ref.shape, scr_ref.dtype)

    for h in range(32):
        piece = x_ref[:, h].reshape(96, tb)
        piece = jnp.transpose(piece).astype(jnp.bfloat16)
        scr_ref[h & 3, h >> 2, :, :96] = piece

    s = [scr_ref[a].reshape(m, _LANE) for a in range(4)]
    s = s + [_roll_up(v, tb) for v in s]

    lhs1 = jnp.concatenate(
        [jnp.concatenate([s[o + kh] for kh in range(5)], axis=1)
         for o in range(4)], axis=0)
    acc1 = jnp.dot(lhs1, c1w_ref[...], preferred_element_type=jnp.float32)
    act1 = jnp.maximum(acc1 + c1b_ref[...], 0.0)
    cand1 = jnp.maximum(act1[:, :_LANE], act1[:, _LANE:])
    y0 = jnp.maximum(cand1[:m], cand1[m:2 * m])
    y1 = jnp.maximum(cand1[2 * m:3 * m], cand1[3 * m:])

    t = [y0.astype(jnp.bfloat16), y1.astype(jnp.bfloat16)]
    t = t + [_roll_up(v, tb) for v in t] + [_roll_up(v, 2 * tb) for v in t]
    lhs2 = jnp.concatenate(
        [jnp.concatenate([t[di + kh] for kh in range(5)], axis=1)
         for di in range(2)], axis=0)
    acc2 = jnp.dot(lhs2, c2w_ref[...], preferred_element_type=jnp.float32)
    act2 = jnp.maximum(acc2 + c2b_ref[...], 0.0)
    cand2 = jnp.maximum(act2[:, :_LANE], act2[:, _LANE:])
    z = jnp.maximum(cand2[:m], cand2[m:]).astype(jnp.bfloat16)

    lhs3 = jnp.concatenate(
        [z[r * tb:(r + 1) * tb] for r in range(5)], axis=1)
    h1 = jnp.dot(lhs3, f1w_ref[...], preferred_element_type=jnp.float32)
    h1 = jnp.maximum(h1 + f1b_ref[...], 0.0).astype(jnp.bfloat16)
    h2 = jnp.dot(h1, f2w_ref[...], preferred_element_type=jnp.float32)
    h2 = jnp.maximum(h2 + f2b_ref[...], 0.0).astype(jnp.bfloat16)
    logits = (jnp.dot(h2, f3w_ref[...], preferred_element_type=jnp.float32)
              + f3b_ref[...])
    o_ref[...] = logits[:, :o_ref.shape[1]]


def _banded_conv(w, w_in, cmajor):
    cout, cin, k, _ = w.shape
    ow = (w_in - k + 1) // 2
    kin, kout = w_in * cin, ow * cout
    kin_p, kout_p = _ceil_to(kin, _LANE), _ceil_to(kout, _LANE)
    halves = []
    for dj in range(2):
        kw = jnp.arange(w_in)[:, None] - 2 * jnp.arange(ow)[None, :] - dj
        ok = ((kw >= 0) & (kw < k))[None, None, None]
        v = w[:, :, :, jnp.clip(kw, 0, k - 1)] * ok
        perm = (2, 1, 3, 4, 0) if cmajor else (2, 3, 1, 4, 0)
        v = jnp.transpose(v, perm).reshape(k, kin, kout)
        halves.append(jnp.pad(v, ((0, 0), (0, kin_p - kin), (0, kout_p - kout))))
    b = jnp.concatenate(halves, axis=2)
    return b.reshape(k * kin_p, 2 * kout_p).astype(jnp.bfloat16)


def _conv_bias(b, ow, kout_p):
    row = jnp.pad(jnp.tile(b, ow), (0, kout_p - ow * b.shape[0]))
    return jnp.concatenate([row, row]).reshape(1, -1).astype(jnp.float32)


def _fc1_banded(w1, oh, ow, cout):
    d_out = w1.shape[0]
    v = w1.reshape(d_out, cout, oh, ow)
    v = jnp.transpose(v, (2, 3, 1, 0)).reshape(oh, ow * cout, d_out)
    v = jnp.pad(v, ((0, 0), (0, _LANE - ow * cout), (0, _LANE - d_out)))
    return v.reshape(oh * _LANE, _LANE).astype(jnp.bfloat16)


def _fc_mat(w):
    return jnp.pad(w.T, ((0, _LANE - w.shape[1]), (0, _LANE - w.shape[0]))
                   ).astype(jnp.bfloat16)


def _fc_bias(b):
    return jnp.pad(b, (0, _LANE - b.shape[0])).reshape(1, -1).astype(jnp.float32)


def kernel(x, conv1_w, conv1_b, conv2_w, conv2_b,
           fc1_w, fc1_b, fc2_w, fc2_b, fc3_w, fc3_b):
    B, C, H, W = x.shape
    bp = _ceil_to(B, _TB)
    if bp != B:
        x = jnp.pad(x, ((0, bp - B), (0, 0), (0, 0), (0, 0)))
    xt = jnp.transpose(x, (1, 2, 3, 0))

    ow1 = (W - 5 + 1) // 2
    ow2 = (ow1 - 5 + 1) // 2

    c1w = _banded_conv(conv1_w, W, cmajor=True)
    c1b = _conv_bias(conv1_b, ow1, _LANE)
    c2w = _banded_conv(conv2_w, ow1, cmajor=False)
    c2b = _conv_bias(conv2_b, ow2, _LANE)
    f1w = _fc1_banded(fc1_w, ow2, ow2, conv2_w.shape[0])
    f1b = _fc_bias(fc1_b)
    f2w = _fc_mat(fc2_w)
    f2b = _fc_bias(fc2_b)
    f3w = _fc_mat(fc3_w)
    f3b = _fc_bias(fc3_b)

    const = lambda i: (0, 0)
    out = pl.pallas_call(
        _lenet_body,
        out_shape=jax.ShapeDtypeStruct((bp, 10), jnp.float32),
        grid=(bp // _TB,),
        in_specs=[
            pl.BlockSpec((C, H, W, _TB), lambda i: (0, 0, 0, i)),
            pl.BlockSpec((5 * _LANE, 2 * _LANE), const),
            pl.BlockSpec((1, 2 * _LANE), const),
            pl.BlockSpec((5 * _LANE, 2 * _LANE), const),
            pl.BlockSpec((1, 2 * _LANE), const),
            pl.BlockSpec((5 * _LANE, _LANE), const),
            pl.BlockSpec((1, _LANE), const),
            pl.BlockSpec((_LANE, _LANE), const),
            pl.BlockSpec((1, _LANE), const),
            pl.BlockSpec((_LANE, _LANE), const),
            pl.BlockSpec((1, _LANE), const),
        ],
        out_specs=pl.BlockSpec((_TB, 10), lambda i: (i, 0)),
        scratch_shapes=[pltpu.VMEM((4, _ROWS, _TB, _LANE), jnp.bfloat16)],
        compiler_params=pltpu.CompilerParams(
            dimension_semantics=("arbitrary",),
            vmem_limit_bytes=48 * 1024 * 1024),
    )(xt, c1w, c1b, c2w, c2b, f1w, f1b, f2w, f2b, f3w, f3b)

    return out[:B]


# speedup vs baseline: 2.2612x; 1.0023x over previous
import jax
import jax.numpy as jnp
from jax.experimental import pallas as pl
from jax.experimental.pallas import tpu as pltpu

_LANE = 128
_ROWS = 8
_TB = 256


def _ceil_to(v, m):
    return (v + m - 1) // m * m


def _roll_up(v, s):
    if s == 0:
        return v
    return jnp.concatenate([v[s:], v[:s]], axis=0)


def _lenet_body(x_ref, wp_ref, bp_ref, o_ref, scr_ref):
    tb = x_ref.shape[3]
    m = tb * _ROWS

    c1w_ref = wp_ref[0:640]
    c2w_ref = wp_ref[640:1280]
    f1w_ref = wp_ref[1280:1920, 0:_LANE]
    f2w_ref = wp_ref[1920:2048, 0:_LANE]
    f3w_ref = wp_ref[2048:2176, 0:_LANE]
    c1b_ref = bp_ref[0:1]
    c2b_ref = bp_ref[1:2]
    f1b_ref = bp_ref[2:3, 0:_LANE]
    f2b_ref = bp_ref[3:4, 0:_LANE]
    f3b_ref = bp_ref[4:5, 0:_LANE]

    @pl.when(pl.program_id(0) == 0)
    def _init():
        scr_ref[...] = jnp.zeros(scr_ref.shape, scr_ref.dtype)

    for h in range(32):
        piece = x_ref[:, h].reshape(96, tb)
        piece = jnp.transpose(piece).astype(jnp.bfloat16)
        scr_ref[h & 3, h >> 2, :, :96] = piece

    s = [scr_ref[a].reshape(m, _LANE) for a in range(4)]
    s = s + [_roll_up(v, tb) for v in s]

    lhs1 = jnp.concatenate(
        [jnp.concatenate([s[o + kh] for kh in range(5)], axis=1)
         for o in range(4)], axis=0)
    acc1 = jnp.dot(lhs1, c1w_ref, preferred_element_type=jnp.float32)
    act1 = jnp.maximum(acc1 + c1b_ref, 0.0)
    cand1 = jnp.maximum(act1[:, :_LANE], act1[:, _LANE:])
    y0 = jnp.maximum(cand1[:m], cand1[m:2 * m])
    y1 = jnp.maximum(cand1[2 * m:3 * m], cand1[3 * m:])

    t = [y0.astype(jnp.bfloat16), y1.astype(jnp.bfloat16)]
    t = t + [_roll_up(v, tb) for v in t] + [_roll_up(v, 2 * tb) for v in t]
    lhs2 = jnp.concatenate(
        [jnp.concatenate([t[di + kh] for kh in range(5)], axis=1)
         for di in range(2)], axis=0)
    acc2 = jnp.dot(lhs2, c2w_ref, preferred_element_type=jnp.float32)
    act2 = jnp.maximum(acc2 + c2b_ref, 0.0)
    cand2 = jnp.maximum(act2[:, :_LANE], act2[:, _LANE:])
    z = jnp.maximum(cand2[:m], cand2[m:]).astype(jnp.bfloat16)

    lhs3 = jnp.concatenate(
        [z[r * tb:(r + 1) * tb] for r in range(5)], axis=1)
    h1 = jnp.dot(lhs3, f1w_ref, preferred_element_type=jnp.float32)
    h1 = jnp.maximum(h1 + f1b_ref, 0.0).astype(jnp.bfloat16)
    h2 = jnp.dot(h1, f2w_ref, preferred_element_type=jnp.float32)
    h2 = jnp.maximum(h2 + f2b_ref, 0.0).astype(jnp.bfloat16)
    logits = (jnp.dot(h2, f3w_ref, preferred_element_type=jnp.float32)
              + f3b_ref)
    o_ref[...] = logits[:, :o_ref.shape[1]]


def _banded_conv(w, w_in, cmajor):
    cout, cin, k, _ = w.shape
    ow = (w_in - k + 1) // 2
    kin, kout = w_in * cin, ow * cout
    kin_p, kout_p = _ceil_to(kin, _LANE), _ceil_to(kout, _LANE)
    halves = []
    for dj in range(2):
        kw = jnp.arange(w_in)[:, None] - 2 * jnp.arange(ow)[None, :] - dj
        ok = ((kw >= 0) & (kw < k))[None, None, None]
        v = w[:, :, :, jnp.clip(kw, 0, k - 1)] * ok
        perm = (2, 1, 3, 4, 0) if cmajor else (2, 3, 1, 4, 0)
        v = jnp.transpose(v, perm).reshape(k, kin, kout)
        halves.append(jnp.pad(v, ((0, 0), (0, kin_p - kin), (0, kout_p - kout))))
    b = jnp.concatenate(halves, axis=2)
    return b.reshape(k * kin_p, 2 * kout_p).astype(jnp.bfloat16)


def _conv_bias(b, ow, kout_p):
    row = jnp.pad(jnp.tile(b, ow), (0, kout_p - ow * b.shape[0]))
    return jnp.concatenate([row, row]).reshape(1, -1).astype(jnp.float32)


def _fc1_banded(w1, oh, ow, cout):
    d_out = w1.shape[0]
    v = w1.reshape(d_out, cout, oh, ow)
    v = jnp.transpose(v, (2, 3, 1, 0)).reshape(oh, ow * cout, d_out)
    v = jnp.pad(v, ((0, 0), (0, _LANE - ow * cout), (0, _LANE - d_out)))
    return v.reshape(oh * _LANE, _LANE).astype(jnp.bfloat16)


def _fc_mat(w):
    return jnp.pad(w.T, ((0, _LANE - w.shape[1]), (0, _LANE - w.shape[0]))
                   ).astype(jnp.bfloat16)


def _fc_bias(b):
    return jnp.pad(b, (0, _LANE - b.shape[0])).reshape(1, -1).astype(jnp.float32)


def kernel(x, conv1_w, conv1_b, conv2_w, conv2_b,
           fc1_w, fc1_b, fc2_w, fc2_b, fc3_w, fc3_b):
    B, C, H, W = x.shape
    bp = _ceil_to(B, _TB)
    if bp != B:
        x = jnp.pad(x, ((0, bp - B), (0, 0), (0, 0), (0, 0)))
    xt = jnp.transpose(x, (1, 2, 3, 0))

    ow1 = (W - 5 + 1) // 2
    ow2 = (ow1 - 5 + 1) // 2

    def _wide(w):
        return jnp.pad(w, ((0, 0), (0, 2 * _LANE - w.shape[1])))

    wpack = jnp.concatenate([
        _banded_conv(conv1_w, W, cmajor=True),
        _banded_conv(conv2_w, ow1, cmajor=False),
        _wide(_fc1_banded(fc1_w, ow2, ow2, conv2_w.shape[0])),
        _wide(_fc_mat(fc2_w)),
        _wide(_fc_mat(fc3_w)),
    ], axis=0)
    bpack = jnp.concatenate([
        _conv_bias(conv1_b, ow1, _LANE),
        _conv_bias(conv2_b, ow2, _LANE),
        _wide(_fc_bias(fc1_b)),
        _wide(_fc_bias(fc2_b)),
        _wide(_fc_bias(fc3_b)),
        jnp.zeros((3, 2 * _LANE), jnp.float32),
    ], axis=0)

    const = lambda i: (0, 0)
    out = pl.pallas_call(
        _lenet_body,
        out_shape=jax.ShapeDtypeStruct((bp, 10), jnp.float32),
        grid=(bp // _TB,),
        in_specs=[
            pl.BlockSpec((C, H, W, _TB), lambda i: (0, 0, 0, i)),
            pl.BlockSpec((17 * _LANE, 2 * _LANE), const),
            pl.BlockSpec((8, 2 * _LANE), const),
        ],
        out_specs=pl.BlockSpec((_TB, 10), lambda i: (i, 0)),
        scratch_shapes=[pltpu.VMEM((4, _ROWS, _TB, _LANE), jnp.bfloat16)],
        compiler_params=pltpu.CompilerParams(
            dimension_semantics=("arbitrary",),
            vmem_limit_bytes=48 * 1024 * 1024),
    )(xt, wpack, bpack)

    return out[:B]
